# Initial kernel scaffold; baseline (speedup 1.0000x reference)
import jax
import jax.numpy as jnp
from jax.experimental import pallas as pl


def kernel(x, edge_index, W1, b1, W2, b2):
    raise NotImplementedError("write your pallas kernel here")


# speedup vs baseline: 49.7266x; 49.7266x over previous
import functools

import jax
import jax.numpy as jnp
from jax import lax
from jax.experimental import pallas as pl
from jax.experimental.pallas import tpu as pltpu
from jax.experimental.pallas import tpu_sc as plsc

N = 10000
D_IN, D_HID, D_OUT = 128, 16, 300
E = 320000

NC, NS = 2, 16
NW = NC * NS
B = 128
KU = 8
NPAD = 10240
RPT = NPAD // NS
EROWS = 2560
EPAD = EROWS * B
RW = EROWS // NW
G = RW // KU

_mesh = plsc.VectorSubcoreMesh(
    core_axis_name="c", subcore_axis_name="s", num_cores=NC, num_subcores=NS)
_sc_params = pltpu.CompilerParams(use_tc_tiling_on_sc=False)


@functools.partial(
    pl.kernel,
    out_type=jax.ShapeDtypeStruct((NC * NPAD,), jnp.float32),
    mesh=_mesh,
    scratch_types=[
        pltpu.VMEM((KU, B), jnp.int32),
        pltpu.VMEM((B,), jnp.float32),
        pltpu.VMEM((RPT,), jnp.float32),
        pltpu.VMEM_SHARED((NPAD,), jnp.float32),
        pltpu.SemaphoreType.DMA,
    ],
    compiler_params=_sc_params,
)
def _deg_kernel(dst_hbm, ones_hbm, zeros_hbm, out_hbm, didx, ones_v, vbuf,
                acc_sh, sem):
    c = lax.axis_index("c")
    s = lax.axis_index("s")
    wid = c * NS + s
    pltpu.sync_copy(ones_hbm, ones_v)
    pltpu.sync_copy(zeros_hbm, vbuf)
    pltpu.sync_copy(vbuf, acc_sh.at[pl.ds(s * RPT, RPT)])
    plsc.subcore_barrier()
    base = wid * RW

    def group(gi, carry):
        row0 = base + gi * KU
        pltpu.sync_copy(dst_hbm.at[pl.ds(row0, KU)], didx)
        descs = [pltpu.async_copy(ones_v, acc_sh.at[didx.at[j]], sem, add=True)
                 for j in range(KU)]
        for d in descs:
            d.wait()
        return carry

    lax.fori_loop(0, G, group, 0)
    plsc.subcore_barrier()
    pltpu.sync_copy(acc_sh.at[pl.ds(s * RPT, RPT)], vbuf)
    pltpu.sync_copy(vbuf, out_hbm.at[pl.ds(c * NPAD + s * RPT, RPT)])


@functools.partial(
    pl.kernel,
    out_type=jax.ShapeDtypeStruct((NC * NPAD, D_HID), jnp.float32),
    mesh=_mesh,
    scratch_types=[
        pltpu.VMEM((KU, B), jnp.int32),
        pltpu.VMEM((KU, B), jnp.int32),
        pltpu.VMEM((KU, B, D_HID), jnp.float32),
        pltpu.VMEM((RPT, D_HID), jnp.float32),
        pltpu.VMEM_SHARED((NPAD, D_HID), jnp.float32),
        pltpu.VMEM_SHARED((NPAD, D_HID), jnp.float32),
        pltpu.SemaphoreType.DMA,
        pltpu.SemaphoreType.DMA,
    ],
    compiler_params=_sc_params,
)
def _prop_kernel(src_hbm, dst_hbm, g_hbm, zeros_hbm, out_hbm, sidx, didx,
                 rows, vbuf, acc_sh, g_sh, gsem, ssem):
    c = lax.axis_index("c")
    s = lax.axis_index("s")
    wid = c * NS + s
    pltpu.sync_copy(zeros_hbm, vbuf)
    pltpu.sync_copy(vbuf, acc_sh.at[pl.ds(s * RPT, RPT)])
    pltpu.sync_copy(g_hbm.at[pl.ds(s * RPT, RPT)], vbuf)
    pltpu.sync_copy(vbuf, g_sh.at[pl.ds(s * RPT, RPT)])
    plsc.subcore_barrier()
    base = wid * RW

    def group(gi, carry):
        row0 = base + gi * KU
        pltpu.sync_copy(src_hbm.at[pl.ds(row0, KU)], sidx)
        pltpu.sync_copy(dst_hbm.at[pl.ds(row0, KU)], didx)
        gd = [pltpu.async_copy(g_sh.at[sidx.at[j]], rows.at[j], gsem)
              for j in range(KU)]
        for d in gd:
            d.wait()
        sd = [pltpu.async_copy(rows.at[j], acc_sh.at[didx.at[j]], ssem,
                               add=True)
              for j in range(KU)]
        for d in sd:
            d.wait()
        return carry

    lax.fori_loop(0, G, group, 0)
    plsc.subcore_barrier()
    pltpu.sync_copy(acc_sh.at[pl.ds(s * RPT, RPT)], vbuf)
    pltpu.sync_copy(vbuf, out_hbm.at[pl.ds(c * NPAD + s * RPT, RPT)])


_BLK1 = 1024


def _tc1_body(x_ref, w_ref, p0_ref, p1_ref, g_ref, dv_ref):
    deg = p0_ref[...] + p1_ref[...] + 1.0
    dinv = lax.rsqrt(deg)
    h = jnp.dot(x_ref[...], w_ref[...], preferred_element_type=jnp.float32)
    g_ref[...] = dinv * h
    dv_ref[...] = jnp.broadcast_to(dinv, (_BLK1, D_HID))


def _tc1(x_p, W1, p0, p1):
    return pl.pallas_call(
        _tc1_body,
        grid=(NPAD // _BLK1,),
        in_specs=[
            pl.BlockSpec((_BLK1, D_IN), lambda i: (i, 0)),
            pl.BlockSpec((D_IN, D_HID), lambda i: (0, 0)),
            pl.BlockSpec((_BLK1, 1), lambda i: (i, 0)),
            pl.BlockSpec((_BLK1, 1), lambda i: (i, 0)),
        ],
        out_specs=[
            pl.BlockSpec((_BLK1, D_HID), lambda i: (i, 0)),
            pl.BlockSpec((_BLK1, D_HID), lambda i: (i, 0)),
        ],
        out_shape=[
            jax.ShapeDtypeStruct((NPAD, D_HID), jnp.float32),
            jax.ShapeDtypeStruct((NPAD, D_HID), jnp.float32),
        ],
    )(x_p, W1, p0, p1)


_BLK2 = 2048


def _tc2_body(a0_ref, a1_ref, g1_ref, dv_ref, b1_ref, g2_ref):
    dv = dv_ref[...]
    z = dv * (a0_ref[...] + a1_ref[...] + g1_ref[...]) + b1_ref[...]
    g2_ref[...] = dv * jnp.maximum(z, 0.0)


def _tc2(a0, a1, g1, dv, b1):
    spec = pl.BlockSpec((_BLK2, D_HID), lambda i: (i, 0))
    return pl.pallas_call(
        _tc2_body,
        grid=(NPAD // _BLK2,),
        in_specs=[spec, spec, spec, spec,
                  pl.BlockSpec((1, D_HID), lambda i: (0, 0))],
        out_specs=spec,
        out_shape=jax.ShapeDtypeStruct((NPAD, D_HID), jnp.float32),
    )(a0, a1, g1, dv, b1)


_BLK3 = 1000


def _tc3_body(q0_ref, q1_ref, g2_ref, dv_ref, w2_ref, b2_ref, out_ref):
    z2 = dv_ref[...] * (q0_ref[...] + q1_ref[...] + g2_ref[...])
    logits = jnp.dot(z2, w2_ref[...],
                     preferred_element_type=jnp.float32) + b2_ref[...]
    mx = jnp.max(logits, axis=1, keepdims=True)
    ex = jnp.exp(logits - mx)
    sm = jnp.sum(ex, axis=1, keepdims=True)
    out_ref[...] = logits - mx - jnp.log(sm)


def _tc3(q0, q1, g2, dv, W2, b2):
    spec16 = pl.BlockSpec((_BLK3, D_HID), lambda i: (i, 0))
    return pl.pallas_call(
        _tc3_body,
        grid=(N // _BLK3,),
        in_specs=[spec16, spec16, spec16, spec16,
                  pl.BlockSpec((D_HID, D_OUT), lambda i: (0, 0)),
                  pl.BlockSpec((1, D_OUT), lambda i: (0, 0))],
        out_specs=pl.BlockSpec((_BLK3, D_OUT), lambda i: (i, 0)),
        out_shape=jax.ShapeDtypeStruct((N, D_OUT), jnp.float32),
    )(q0, q1, g2, dv, W2, b2)


def kernel(x, edge_index, W1, b1, W2, b2):
    src = edge_index[0].astype(jnp.int32)
    dst = edge_index[1].astype(jnp.int32)
    pad = jnp.full((EPAD - E,), N, jnp.int32)
    src2 = jnp.concatenate([src, pad]).reshape(EROWS, B)
    dst2 = jnp.concatenate([dst, pad]).reshape(EROWS, B)
    ones_b = jnp.ones((B,), jnp.float32)
    zeros_d = jnp.zeros((RPT,), jnp.float32)
    zeros16 = jnp.zeros((RPT, D_HID), jnp.float32)
    x_p = jnp.pad(x, ((0, NPAD - N), (0, 0)))

    degp = _deg_kernel(dst2, ones_b, zeros_d)
    p0 = degp[:NPAD].reshape(NPAD, 1)
    p1 = degp[NPAD:].reshape(NPAD, 1)
    g1, dv = _tc1(x_p, W1, p0, p1)

    parts = _prop_kernel(src2, dst2, g1, zeros16)
    g2 = _tc2(parts[:NPAD], parts[NPAD:], g1, dv, b1.reshape(1, D_HID))

    parts2 = _prop_kernel(src2, dst2, g2, zeros16)
    return _tc3(parts2[:NPAD], parts2[NPAD:], g2, dv, W2,
                b2.reshape(1, D_OUT))


# speedup vs baseline: 60.7853x; 1.2224x over previous
import functools

import jax
import jax.numpy as jnp
from jax import lax
from jax.experimental import pallas as pl
from jax.experimental.pallas import tpu as pltpu
from jax.experimental.pallas import tpu_sc as plsc

N = 10000
D_IN, D_HID, D_OUT = 128, 16, 300
E = 320000

NC, NS = 2, 16
NW = NC * NS
B = 128
EROWS = E // B
RW = EROWS // NW
REM = EROWS - RW * NW
KU = 6
G = RW // KU
NPAD = 10240
RPT = NPAD // NS
DPT = NPAD // NW

_mesh = plsc.VectorSubcoreMesh(
    core_axis_name="c", subcore_axis_name="s", num_cores=NC, num_subcores=NS)
_sc_params = pltpu.CompilerParams(use_tc_tiling_on_sc=False,
                                  needs_layout_passes=False)


def _newton_rsqrt(d):
    i = plsc.bitcast(d, jnp.int32)
    y = plsc.bitcast(jnp.full((16,), 0x5F3759DF, jnp.int32)
                     - (i >> jnp.full((16,), 1, jnp.int32)), jnp.float32)
    for _ in range(3):
        y = y * (1.5 - 0.5 * d * y * y)
    return y


_DEG_RW = EROWS // NS
_DEG_REM = EROWS - _DEG_RW * NS
_DEG_G = 26


@functools.partial(
    pl.kernel,
    out_type=jax.ShapeDtypeStruct((NPAD, D_HID), jnp.float32),
    mesh=_mesh,
    scratch_types=[
        pltpu.VMEM((KU, B), jnp.int32),
        pltpu.VMEM((B,), jnp.float32),
        pltpu.VMEM((RPT,), jnp.float32),
        pltpu.VMEM((DPT, D_HID), jnp.float32),
        pltpu.VMEM_SHARED((NPAD,), jnp.float32),
        pltpu.SemaphoreType.DMA,
    ],
    compiler_params=_sc_params,
)
def _deg_kernel(dst_hbm, ones_hbm, zeros_hbm, out_hbm, didx, ones_v, vbuf,
                dvbuf, acc_sh, sem):
    c = lax.axis_index("c")
    s = lax.axis_index("s")
    pltpu.sync_copy(ones_hbm, ones_v)
    pltpu.sync_copy(zeros_hbm, vbuf)
    pltpu.sync_copy(vbuf, acc_sh.at[pl.ds(s * RPT, RPT)])
    plsc.subcore_barrier()
    base = s * _DEG_RW

    def group(gi, carry):
        row0 = base + gi * KU
        pltpu.sync_copy(dst_hbm.at[pl.ds(row0, KU)], didx)
        descs = [pltpu.async_copy(ones_v, acc_sh.at[didx.at[j]], sem, add=True)
                 for j in range(KU)]
        for d in descs:
            d.wait()
        return carry

    lax.fori_loop(0, _DEG_G, group, 0)

    @pl.when(s < _DEG_REM)
    def _():
        pltpu.sync_copy(dst_hbm.at[pl.ds(NS * _DEG_RW + s, 1)],
                        didx.at[pl.ds(0, 1)])
        pltpu.async_copy(ones_v, acc_sh.at[didx.at[0]], sem, add=True).wait()

    plsc.subcore_barrier()
    off = (c * NS + s) * DPT
    pltpu.sync_copy(acc_sh.at[pl.ds(off, DPT)], vbuf.at[pl.ds(0, DPT)])

    def chunk(k, carry):
        d = vbuf[pl.ds(k * 16, 16)] + 1.0
        y = _newton_rsqrt(d)
        for j in range(16):
            dvbuf[k * 16 + j] = jnp.full((16,), y[j], jnp.float32)
        return carry

    lax.fori_loop(0, DPT // 16, chunk, 0)
    pltpu.sync_copy(dvbuf, out_hbm.at[pl.ds(off, DPT)])


def _make_prop(layer0):
    if layer0:
        extra_in = [
            pltpu.VMEM((RPT, D_HID), jnp.float32),
        ]
    else:
        extra_in = [
            pltpu.VMEM((RPT, D_HID), jnp.float32),
            pltpu.VMEM((RPT, D_HID), jnp.float32),
            pltpu.VMEM((D_HID,), jnp.float32),
        ]

    @functools.partial(
        pl.kernel,
        out_type=jax.ShapeDtypeStruct((NC * NPAD, D_HID), jnp.float32),
        mesh=_mesh,
        scratch_types=[
            pltpu.VMEM((KU, B), jnp.int32),
            pltpu.VMEM((KU, B), jnp.int32),
            pltpu.VMEM((KU, B, D_HID), jnp.float32),
            pltpu.VMEM((RPT, D_HID), jnp.float32),
            pltpu.VMEM((RPT, D_HID), jnp.float32),
            pltpu.VMEM((RPT, D_HID), jnp.float32),
        ] + extra_in + [
            pltpu.VMEM_SHARED((NPAD, D_HID), jnp.float32),
            pltpu.VMEM_SHARED((NPAD, D_HID), jnp.float32),
            pltpu.SemaphoreType.DMA,
            pltpu.SemaphoreType.DMA,
        ],
        compiler_params=_sc_params,
    )
    def _prop(src_hbm, dst_hbm, dv_hbm, feat_hbm, b1_hbm, out_hbm, sidx, didx,
              rows, dvbuf, gbuf, abuf, *rest):
        if layer0:
            (hbuf, acc_sh, g_sh, gsem, ssem) = rest
        else:
            (a0buf, a1buf, b1buf, acc_sh, g_sh, gsem, ssem) = rest
        c = lax.axis_index("c")
        s = lax.axis_index("s")
        wid = c * NS + s
        off = s * RPT
        pltpu.sync_copy(dv_hbm.at[pl.ds(off, RPT)], dvbuf)
        if layer0:
            pltpu.sync_copy(feat_hbm.at[pl.ds(off, RPT)], hbuf)
        else:
            pltpu.sync_copy(feat_hbm.at[pl.ds(off, RPT)], a0buf)
            pltpu.sync_copy(feat_hbm.at[pl.ds(NPAD + off, RPT)], a1buf)
            pltpu.sync_copy(b1_hbm, b1buf)
        cmask = jnp.where(c == 0, jnp.float32(1.0), jnp.float32(0.0))

        def stage(i, carry):
            dv = dvbuf[i]
            if layer0:
                g = dv * hbuf[i]
            else:
                z = dv * (a0buf[i] + a1buf[i]) + b1buf[...]
                g = dv * jnp.maximum(z, 0.0)
            gbuf[i] = g
            abuf[i] = g * cmask
            return carry

        lax.fori_loop(0, RPT, stage, 0)
        pltpu.sync_copy(gbuf, g_sh.at[pl.ds(off, RPT)])
        pltpu.sync_copy(abuf, acc_sh.at[pl.ds(off, RPT)])
        plsc.subcore_barrier()

        base = wid * RW

        def group(gi, carry):
            row0 = base + gi * KU
            pltpu.sync_copy(src_hbm.at[pl.ds(row0, KU)], sidx)
            pltpu.sync_copy(dst_hbm.at[pl.ds(row0, KU)], didx)
            gd = [pltpu.async_copy(g_sh.at[sidx.at[j]], rows.at[j], gsem)
                  for j in range(KU)]
            sd = []
            for j in range(KU):
                gd[j].wait()
                sd.append(pltpu.async_copy(rows.at[j], acc_sh.at[didx.at[j]],
                                           ssem, add=True))
            for d in sd:
                d.wait()
            return carry

        lax.fori_loop(0, G, group, 0)

        @pl.when(wid < REM)
        def _():
            row0 = NW * RW + wid
            pltpu.sync_copy(src_hbm.at[pl.ds(row0, 1)], sidx.at[pl.ds(0, 1)])
            pltpu.sync_copy(dst_hbm.at[pl.ds(row0, 1)], didx.at[pl.ds(0, 1)])
            pltpu.async_copy(g_sh.at[sidx.at[0]], rows.at[0], gsem).wait()
            pltpu.async_copy(rows.at[0], acc_sh.at[didx.at[0]], ssem,
                             add=True).wait()

        plsc.subcore_barrier()
        pltpu.sync_copy(acc_sh.at[pl.ds(off, RPT)], abuf)
        pltpu.sync_copy(abuf, out_hbm.at[pl.ds(c * NPAD + off, RPT)])

    return _prop


_prop1 = _make_prop(True)
_prop2 = _make_prop(False)


_BLK = 1000


def _t0_body(x_ref, w_ref, h_ref):
    h_ref[...] = jnp.dot(x_ref[...], w_ref[...],
                         preferred_element_type=jnp.float32)


def _t0(x, W1):
    return pl.pallas_call(
        _t0_body,
        grid=(N // _BLK,),
        in_specs=[
            pl.BlockSpec((_BLK, D_IN), lambda i: (i, 0)),
            pl.BlockSpec((D_IN, D_HID), lambda i: (0, 0)),
        ],
        out_specs=pl.BlockSpec((_BLK, D_HID), lambda i: (i, 0)),
        out_shape=jax.ShapeDtypeStruct((NPAD, D_HID), jnp.float32),
    )(x, W1)


def _t3_body(q0_ref, q1_ref, dv_ref, w2_ref, b2_ref, out_ref):
    z2 = dv_ref[...] * (q0_ref[0] + q1_ref[0])
    logits = jnp.dot(z2, w2_ref[...],
                     preferred_element_type=jnp.float32) + b2_ref[...]
    mx = jnp.max(logits, axis=1, keepdims=True)
    ex = jnp.exp(logits - mx)
    sm = jnp.sum(ex, axis=1, keepdims=True)
    out_ref[...] = logits - mx - jnp.log(sm)


def _t3(parts, dv, W2, b2):
    return pl.pallas_call(
        _t3_body,
        grid=(N // _BLK,),
        in_specs=[
            pl.BlockSpec((1, _BLK, D_HID), lambda i: (0, i, 0)),
            pl.BlockSpec((1, _BLK, D_HID), lambda i: (1, i, 0)),
            pl.BlockSpec((_BLK, D_HID), lambda i: (i, 0)),
            pl.BlockSpec((D_HID, D_OUT), lambda i: (0, 0)),
            pl.BlockSpec((1, D_OUT), lambda i: (0, 0)),
        ],
        out_specs=pl.BlockSpec((_BLK, D_OUT), lambda i: (i, 0)),
        out_shape=jax.ShapeDtypeStruct((N, D_OUT), jnp.float32),
    )(parts, parts, dv, W2, b2.reshape(1, D_OUT))


def kernel(x, edge_index, W1, b1, W2, b2):
    src2 = edge_index[0].astype(jnp.int32).reshape(EROWS, B)
    dst2 = edge_index[1].astype(jnp.int32).reshape(EROWS, B)
    ones_b = jnp.ones((B,), jnp.float32)
    zeros_d = jnp.zeros((RPT,), jnp.float32)

    h = _t0(x, W1)
    dv = _deg_kernel(dst2, ones_b, zeros_d)
    parts = _prop1(src2, dst2, dv, h, jnp.zeros((D_HID,), jnp.float32))
    parts2 = _prop2(src2, dst2, dv, parts, b1)
    return _t3(parts2.reshape(NC, NPAD, D_HID), dv, W2, b2)


# speedup vs baseline: 80.3793x; 1.3223x over previous
import functools

import jax
import jax.numpy as jnp
from jax import lax
from jax.experimental import pallas as pl
from jax.experimental.pallas import tpu as pltpu
from jax.experimental.pallas import tpu_sc as plsc

N = 10000
D_IN, D_HID, D_OUT = 128, 16, 300
E = 320000

NC, NS = 2, 16
NW = NC * NS
B = 256
EROWS = E // B
RW = EROWS // NW
REM = EROWS - RW * NW
KU = 3
G = RW // KU
NPAD = 10240
RPT = NPAD // NS
DPT = NPAD // NW

_mesh = plsc.VectorSubcoreMesh(
    core_axis_name="c", subcore_axis_name="s", num_cores=NC, num_subcores=NS)
_sc_params = pltpu.CompilerParams(use_tc_tiling_on_sc=False,
                                  needs_layout_passes=False)


def _newton_rsqrt(d):
    i = plsc.bitcast(d, jnp.int32)
    y = plsc.bitcast(jnp.full((16,), 0x5F3759DF, jnp.int32)
                     - (i >> jnp.full((16,), 1, jnp.int32)), jnp.float32)
    for _ in range(3):
        y = y * (1.5 - 0.5 * d * y * y)
    return y


@functools.partial(
    pl.kernel,
    out_type=jax.ShapeDtypeStruct((NC * NPAD,), jnp.float32),
    mesh=_mesh,
    scratch_types=[
        pltpu.VMEM((2, KU, B), jnp.int32),
        pltpu.VMEM((B,), jnp.float32),
        pltpu.VMEM((RPT,), jnp.float32),
        pltpu.VMEM_SHARED((NPAD,), jnp.float32),
        pltpu.SemaphoreType.DMA,
    ],
    compiler_params=_sc_params,
)
def _deg_kernel(eidx_hbm, ones_hbm, zeros_hbm, out_hbm, didx, ones_v, vbuf,
                acc_sh, sem):
    c = lax.axis_index("c")
    s = lax.axis_index("s")
    wid = c * NS + s
    pltpu.sync_copy(ones_hbm, ones_v)
    pltpu.sync_copy(zeros_hbm, vbuf)
    pltpu.sync_copy(vbuf, acc_sh.at[pl.ds(s * RPT, RPT)])
    plsc.subcore_barrier()
    base = wid * RW

    def pair(k, carry):
        row0 = base + (2 * k) * KU
        pltpu.sync_copy(eidx_hbm.at[1].at[pl.ds(row0, KU)], didx.at[0])
        da = [pltpu.async_copy(ones_v, acc_sh.at[didx.at[0].at[j]], sem,
                               add=True) for j in range(KU)]
        pltpu.sync_copy(eidx_hbm.at[1].at[pl.ds(row0 + KU, KU)], didx.at[1])
        db = [pltpu.async_copy(ones_v, acc_sh.at[didx.at[1].at[j]], sem,
                               add=True) for j in range(KU)]
        for d in da + db:
            d.wait()
        return carry

    lax.fori_loop(0, G // 2, pair, 0)
    pltpu.sync_copy(eidx_hbm.at[1].at[pl.ds(base + (G - 1) * KU, KU)],
                    didx.at[0])
    dd = [pltpu.async_copy(ones_v, acc_sh.at[didx.at[0].at[j]], sem,
                           add=True) for j in range(KU)]
    for d in dd:
        d.wait()

    @pl.when(wid < REM)
    def _():
        pltpu.sync_copy(eidx_hbm.at[1].at[pl.ds(NW * RW + wid, 1)],
                        didx.at[0].at[pl.ds(0, 1)])
        pltpu.async_copy(ones_v, acc_sh.at[didx.at[0].at[0]], sem,
                         add=True).wait()

    plsc.subcore_barrier()
    pltpu.sync_copy(acc_sh.at[pl.ds(s * RPT, RPT)], vbuf)
    pltpu.sync_copy(vbuf, out_hbm.at[pl.ds(c * NPAD + s * RPT, RPT)])


def _make_prop(layer0):
    if layer0:
        extra_in = [
            pltpu.VMEM((RPT, D_HID), jnp.float32),
        ]
    else:
        extra_in = [
            pltpu.VMEM((RPT, D_HID), jnp.float32),
            pltpu.VMEM((RPT, D_HID), jnp.float32),
            pltpu.VMEM((D_HID,), jnp.float32),
        ]
    extra_in += [
        pltpu.VMEM((RPT,), jnp.float32),
        pltpu.VMEM((RPT,), jnp.float32),
    ]

    @functools.partial(
        pl.kernel,
        out_type=jax.ShapeDtypeStruct((NC * NPAD, D_HID), jnp.float32),
        mesh=_mesh,
        scratch_types=[
            pltpu.VMEM((2, KU, B), jnp.int32),
            pltpu.VMEM((2, KU, B), jnp.int32),
            pltpu.VMEM((2, KU, B, D_HID), jnp.float32),
            pltpu.VMEM((RPT, D_HID), jnp.float32),
            pltpu.VMEM((RPT, D_HID), jnp.float32),
            pltpu.VMEM((RPT, D_HID), jnp.float32),
        ] + extra_in + [
            pltpu.VMEM_SHARED((NPAD, D_HID), jnp.float32),
            pltpu.VMEM_SHARED((NPAD, D_HID), jnp.float32),
            pltpu.SemaphoreType.DMA,
            pltpu.SemaphoreType.DMA,
        ],
        compiler_params=_sc_params,
    )
    def _prop(eidx_hbm, degp_hbm, feat_hbm, b1_hbm, out_hbm, sidx, didx,
              rows, dvbuf, gbuf, abuf, *rest):
        if layer0:
            (hbuf, p0buf, p1buf, acc_sh, g_sh, gsem, ssem) = rest
        else:
            (a0buf, a1buf, b1buf, p0buf, p1buf, acc_sh, g_sh, gsem,
             ssem) = rest
        c = lax.axis_index("c")
        s = lax.axis_index("s")
        wid = c * NS + s
        off = s * RPT
        pltpu.sync_copy(degp_hbm.at[pl.ds(off, RPT)], p0buf)
        pltpu.sync_copy(degp_hbm.at[pl.ds(NPAD + off, RPT)], p1buf)
        if layer0:
            pltpu.sync_copy(feat_hbm.at[pl.ds(off, RPT)], hbuf)
        else:
            pltpu.sync_copy(feat_hbm.at[pl.ds(off, RPT)], a0buf)
            pltpu.sync_copy(feat_hbm.at[pl.ds(NPAD + off, RPT)], a1buf)
            pltpu.sync_copy(b1_hbm, b1buf)

        def dvchunk(k, carry):
            d = p0buf[pl.ds(k * 16, 16)] + p1buf[pl.ds(k * 16, 16)] + 1.0
            y = _newton_rsqrt(d)
            for j in range(16):
                dvbuf[k * 16 + j] = jnp.full((16,), y[j], jnp.float32)
            return carry

        lax.fori_loop(0, RPT // 16, dvchunk, 0)
        cmask = jnp.where(c == 0, jnp.float32(1.0), jnp.float32(0.0))

        def stage(i2, carry):
            for u in range(4):
                i = i2 * 4 + u
                dv = dvbuf[i]
                if layer0:
                    g = dv * hbuf[i]
                else:
                    z = a0buf[i] + a1buf[i] + b1buf[...]
                    g = dv * jnp.maximum(z, 0.0)
                gbuf[i] = g
                abuf[i] = g * cmask
            return carry

        lax.fori_loop(0, RPT // 4, stage, 0)
        pltpu.sync_copy(gbuf, g_sh.at[pl.ds(off, RPT)])
        pltpu.sync_copy(abuf, acc_sh.at[pl.ds(off, RPT)])
        plsc.subcore_barrier()

        base = wid * RW

        def _load_idx(row0, p):
            pltpu.sync_copy(eidx_hbm.at[0].at[pl.ds(row0, KU)], sidx.at[p])
            pltpu.sync_copy(eidx_hbm.at[1].at[pl.ds(row0, KU)], didx.at[p])

        def _gather(p):
            return [pltpu.async_copy(g_sh.at[sidx.at[p].at[j]],
                                     rows.at[p].at[j], gsem)
                    for j in range(KU)]

        def _scatter_interleaved(gd, p):
            sd = []
            for j in range(KU):
                gd[j].wait()
                sd.append(pltpu.async_copy(rows.at[p].at[j],
                                           acc_sh.at[didx.at[p].at[j]],
                                           ssem, add=True))
            return sd

        def pair(k, carry):
            row_a = base + (2 * k) * KU
            _load_idx(row_a, 0)
            gd_a = _gather(0)
            _load_idx(row_a + KU, 1)
            sd_a = _scatter_interleaved(gd_a, 0)
            gd_b = _gather(1)
            sd_b = _scatter_interleaved(gd_b, 1)
            for d in sd_a:
                d.wait()
            for d in sd_b:
                d.wait()
            return carry

        lax.fori_loop(0, G // 2, pair, 0)
        _load_idx(base + (G - 1) * KU, 0)
        sd = _scatter_interleaved(_gather(0), 0)
        for d in sd:
            d.wait()

        @pl.when(wid < REM)
        def _():
            row0 = NW * RW + wid
            pltpu.sync_copy(eidx_hbm.at[0].at[pl.ds(row0, 1)],
                            sidx.at[0].at[pl.ds(0, 1)])
            pltpu.sync_copy(eidx_hbm.at[1].at[pl.ds(row0, 1)],
                            didx.at[0].at[pl.ds(0, 1)])
            pltpu.async_copy(g_sh.at[sidx.at[0].at[0]], rows.at[0].at[0],
                             gsem).wait()
            pltpu.async_copy(rows.at[0].at[0], acc_sh.at[didx.at[0].at[0]],
                             ssem, add=True).wait()

        plsc.subcore_barrier()
        pltpu.sync_copy(acc_sh.at[pl.ds(off, RPT)], abuf)

        def premult(i2, carry):
            for u in range(4):
                i = i2 * 4 + u
                abuf[i] = dvbuf[i] * abuf[i]
            return carry

        lax.fori_loop(0, RPT // 4, premult, 0)
        pltpu.sync_copy(abuf, out_hbm.at[pl.ds(c * NPAD + off, RPT)])

    return _prop


_prop1 = _make_prop(True)
_prop2 = _make_prop(False)


_BLK = 1024
_HROWS = NPAD // 8
_QBLK = 128


def _t0_body(x_ref, w_ref, h_ref):
    h_ref[...] = jnp.dot(x_ref[...], w_ref[...],
                         preferred_element_type=jnp.float32)


def _t0(x, W1):
    return pl.pallas_call(
        _t0_body,
        grid=(NPAD // _BLK,),
        in_specs=[
            pl.BlockSpec((_BLK, D_IN), lambda i: (i, 0)),
            pl.BlockSpec((D_IN, D_HID), lambda i: (0, 0)),
        ],
        out_specs=pl.BlockSpec((_BLK, D_HID), lambda i: (i, 0)),
        out_shape=jax.ShapeDtypeStruct((NPAD, D_HID), jnp.float32),
    )(x, W1)


def _t3_body(q0_ref, q1_ref, w2t_ref, b2_ref, out_ref):
    z2 = q0_ref[0] + q1_ref[0]
    z2t = jnp.transpose(z2, (1, 0))
    logits = jnp.dot(w2t_ref[...], z2t,
                     preferred_element_type=jnp.float32) + b2_ref[...]
    mx = jnp.max(logits, axis=0, keepdims=True)
    ex = jnp.exp(logits - mx)
    sm = jnp.sum(ex, axis=0, keepdims=True)
    out_ref[...] = logits - mx - jnp.log(sm)


def _t3(parts, W2t, b2):
    return pl.pallas_call(
        _t3_body,
        grid=(NPAD // _BLK,),
        in_specs=[
            pl.BlockSpec((1, _BLK, D_HID), lambda i: (0, i, 0)),
            pl.BlockSpec((1, _BLK, D_HID), lambda i: (1, i, 0)),
            pl.BlockSpec((D_OUT, D_HID), lambda i: (0, 0)),
            pl.BlockSpec((D_OUT, 1), lambda i: (0, 0)),
        ],
        out_specs=pl.BlockSpec((D_OUT, _BLK), lambda i: (0, i)),
        out_shape=jax.ShapeDtypeStruct((D_OUT, N), jnp.float32),
    )(parts, parts, W2t, b2.reshape(D_OUT, 1))


def kernel(x, edge_index, W1, b1, W2, b2):
    eidx = edge_index.astype(jnp.int32).reshape(2, EROWS, B)
    ones_b = jnp.ones((B,), jnp.float32)
    zeros_d = jnp.zeros((RPT,), jnp.float32)

    h = _t0(x, W1)
    degp = _deg_kernel(eidx, ones_b, zeros_d)
    parts = _prop1(eidx, degp, h, jnp.zeros((D_HID,), jnp.float32))
    parts2 = _prop2(eidx, degp, parts, b1)
    out_t = _t3(parts2.reshape(NC, NPAD, D_HID), W2.T, b2)
    return out_t.T
